# Initial kernel scaffold; baseline (speedup 1.0000x reference)
import jax
import jax.numpy as jnp
from jax.experimental import pallas as pl


def kernel(inputs, lookup_table):
    raise NotImplementedError("write your pallas kernel here")


# speedup vs baseline: 1.3427x; 1.3427x over previous
import functools

import jax
import jax.numpy as jnp
from jax.experimental import pallas as pl

_BLK = 512


def _vq_kernel(z_ref, c_ref, k_ref, zq_ref):
    z = z_ref[...]
    c = c_ref[...]
    dots = jax.lax.dot_general(
        z, c, (((1,), (1,)), ((), ())),
        preferred_element_type=jnp.float32)
    z_sq = jnp.sum(z * z, axis=1, keepdims=True)
    c_sq = jnp.sum(c * c, axis=1)
    score = (z_sq - 2.0 * dots) + c_sq[None, :]
    idx = jnp.argmin(score, axis=1).astype(jnp.int32)
    k_ref[0, 0, :] = idx
    onehot = (jax.lax.broadcasted_iota(jnp.int32, score.shape, 1)
              == idx[:, None]).astype(jnp.float32)
    zq_ref[...] = jax.lax.dot_general(
        onehot, c, (((1,), (0,)), ((), ())),
        preferred_element_type=jnp.float32)


@functools.partial(jax.jit, static_argnums=())
def kernel(inputs, lookup_table):
    B, H, W, D = inputs.shape
    N = B * H * W
    K = lookup_table.shape[0]
    z = inputs.reshape(N, D)
    nblk = N // _BLK
    k3, zq = pl.pallas_call(
        _vq_kernel,
        grid=(nblk,),
        in_specs=[
            pl.BlockSpec((_BLK, D), lambda i: (i, 0)),
            pl.BlockSpec((K, D), lambda i: (0, 0)),
        ],
        out_specs=[
            pl.BlockSpec((1, 1, _BLK), lambda i: (i, 0, 0)),
            pl.BlockSpec((_BLK, D), lambda i: (i, 0)),
        ],
        out_shape=[
            jax.ShapeDtypeStruct((nblk, 1, _BLK), jnp.int32),
            jax.ShapeDtypeStruct((N, D), jnp.float32),
        ],
    )(z, lookup_table)
    k = k3.reshape(B, H, W)
    z_q = zq.reshape(B, H, W, D)
    return (k, z_q)


# speedup vs baseline: 1.9681x; 1.4657x over previous
import functools

import jax
import jax.numpy as jnp
from jax import lax
from jax.experimental import pallas as pl
from jax.experimental.pallas import tpu as pltpu
from jax.experimental.pallas import tpu_sc as plsc

_BLK = 512


def _vq_argmin_kernel(z_ref, c_ref, k_ref, csq_ref):
    @pl.when(pl.program_id(0) == 0)
    def _():
        c0 = c_ref[...]
        csq_ref[...] = jnp.sum(c0 * c0, axis=1)[None, :]

    z = z_ref[...]
    c = c_ref[...]
    dots = jax.lax.dot_general(
        z, c, (((1,), (1,)), ((), ())),
        preferred_element_type=jnp.float32)
    z_sq = jnp.sum(z * z, axis=1, keepdims=True)
    score = (z_sq - 2.0 * dots) + csq_ref[...]
    k_ref[0, 0, :] = jnp.argmin(score, axis=1).astype(jnp.int32)


def _make_sc_gather(N, nw):
    bpw = N // nw
    nchunk = bpw // 128
    mesh = plsc.VectorSubcoreMesh(core_axis_name="c", subcore_axis_name="s")

    @functools.partial(
        pl.kernel,
        out_type=jax.ShapeDtypeStruct((N, 128), jnp.float32),
        mesh=mesh,
        scratch_types=[
            pltpu.VMEM((nchunk, 128), jnp.int32),
            pltpu.VMEM((bpw, 128), jnp.float32),
            pltpu.SemaphoreType.DMA,
        ],
    )
    def _gather(table_hbm, idx_hbm, out_hbm, idx_v, rows_v, sem):
        nc = plsc.get_sparse_core_info().num_cores
        wid = lax.axis_index("s") * nc + lax.axis_index("c")
        base = wid * bpw
        pltpu.sync_copy(idx_hbm.at[pl.ds(wid * nchunk, nchunk)], idx_v)
        copies = [
            pltpu.async_copy(
                table_hbm.at[idx_v.at[j]],
                rows_v.at[pl.ds(j * 128, 128)], sem)
            for j in range(nchunk)
        ]
        for cp in copies:
            cp.wait()
        pltpu.sync_copy(rows_v, out_hbm.at[pl.ds(base, bpw)])

    return _gather


@jax.jit
def kernel(inputs, lookup_table):
    B, H, W, D = inputs.shape
    N = B * H * W
    K = lookup_table.shape[0]
    z = inputs.reshape(N, D)
    nblk = N // _BLK
    k3 = pl.pallas_call(
        _vq_argmin_kernel,
        grid=(nblk,),
        in_specs=[
            pl.BlockSpec((_BLK, D), lambda i: (i, 0)),
            pl.BlockSpec((K, D), lambda i: (0, 0)),
        ],
        out_specs=pl.BlockSpec((1, 1, _BLK), lambda i: (i, 0, 0)),
        out_shape=jax.ShapeDtypeStruct((nblk, 1, _BLK), jnp.int32),
        scratch_shapes=[pltpu.VMEM((1, K), jnp.float32)],
    )(z, lookup_table)
    idx2 = k3.reshape(N // 128, 128)

    info = plsc.get_sparse_core_info()
    nw = info.num_cores * info.num_subcores
    table_pad = jnp.pad(lookup_table, ((0, 0), (0, 128 - D)))
    zq128 = _make_sc_gather(N, nw)(table_pad, idx2)
    zq = zq128[:, :D]

    return (k3.reshape(B, H, W), zq.reshape(B, H, W, D))
